# Initial kernel scaffold; baseline (speedup 1.0000x reference)
import jax
import jax.numpy as jnp
from jax.experimental import pallas as pl


def kernel(x, w1_sh, w2_sh, w1_ex, w2_ex, w_router):
    raise NotImplementedError("write your pallas kernel here")


# speedup vs baseline: 1.3570x; 1.3570x over previous
import functools

import jax
import jax.numpy as jnp
from jax.experimental import pallas as pl
from jax.experimental.pallas import tpu as pltpu

N_EXP = 8
N_SLOTS = 16
TBLK = 512


def _gelu(v):
    return jax.nn.gelu(v, approximate=True)


def _router_kernel(x_ref, wr_ref, gates_ref):
    x = x_ref[...]
    wr = wr_ref[...]
    logits = jnp.dot(x, wr.T, preferred_element_type=jnp.float32)
    null = logits[:, N_EXP:N_EXP + 1]
    lane = jax.lax.broadcasted_iota(jnp.int32, logits.shape, 1)
    expanded = jnp.where(lane < N_EXP, logits, null)
    v1 = jnp.max(expanded, axis=1, keepdims=True)
    i1 = jnp.min(jnp.where(expanded == v1, lane, N_SLOTS), axis=1, keepdims=True)
    masked = jnp.where(lane == i1, -jnp.inf, expanded)
    v2 = jnp.max(masked, axis=1, keepdims=True)
    i2 = jnp.min(jnp.where(masked == v2, lane, N_SLOTS), axis=1, keepdims=True)
    e2 = jnp.exp(v2 - v1)
    denom = 1.0 + e2
    g1 = 1.0 / denom
    g2 = e2 / denom
    r1 = (i1 < N_EXP).astype(jnp.float32)
    r2 = (i2 < N_EXP).astype(jnp.float32)
    rs = jnp.clip(g1 * r1 + g2 * r2, 1e-9, None)
    has = ((r1 + r2) > 0).astype(jnp.float32)
    rn1 = g1 * r1 / rs * has
    rn2 = g2 * r2 / rs * has
    gates = jnp.where(lane == i1, rn1, 0.0) + jnp.where(lane == i2, rn2, 0.0)
    gates_ref[...] = gates


def _ffn_kernel(x_ref, gates_ref, w1sh_ref, w2sh_ref, w1e_ref, w2e_ref, out_ref):
    e = pl.program_id(1)
    x = x_ref[...]

    @pl.when(e == 0)
    def _():
        h_sh = _gelu(jnp.dot(x, w1sh_ref[...].T, preferred_element_type=jnp.float32))
        out_ref[...] = jnp.dot(h_sh, w2sh_ref[...].T, preferred_element_type=jnp.float32)

    w1 = w1e_ref[0]
    w2 = w2e_ref[0]
    h = _gelu(jnp.dot(x, w1.T, preferred_element_type=jnp.float32))
    y = jnp.dot(h, w2.T, preferred_element_type=jnp.float32)
    gates = gates_ref[...]
    lane = jax.lax.broadcasted_iota(jnp.int32, gates.shape, 1)
    g = jnp.sum(jnp.where(lane == e, gates, 0.0), axis=1, keepdims=True)
    out_ref[...] += g * y


@functools.partial(jax.jit, static_argnames=("interpret",))
def kernel(x, w1_sh, w2_sh, w1_ex, w2_ex, w_router, interpret=False):
    Bv, Tv, Dv = x.shape
    tok = Bv * Tv
    n_exp, h_ex, _ = w1_ex.shape
    h_sh = w1_sh.shape[0]
    assert n_exp == N_EXP
    xf = x.reshape(tok, Dv)
    wr_pad = jnp.zeros((N_SLOTS, Dv), jnp.float32).at[: N_EXP + 1].set(w_router)
    tb = tok // TBLK

    gates = pl.pallas_call(
        _router_kernel,
        grid=(tb,),
        in_specs=[
            pl.BlockSpec((TBLK, Dv), lambda i: (i, 0)),
            pl.BlockSpec((N_SLOTS, Dv), lambda i: (0, 0)),
        ],
        out_specs=pl.BlockSpec((TBLK, N_SLOTS), lambda i: (i, 0)),
        out_shape=jax.ShapeDtypeStruct((tok, N_SLOTS), jnp.float32),
        interpret=interpret,
    )(xf, wr_pad)

    out = pl.pallas_call(
        _ffn_kernel,
        grid=(tb, N_EXP),
        in_specs=[
            pl.BlockSpec((TBLK, Dv), lambda i, e: (i, 0)),
            pl.BlockSpec((TBLK, N_SLOTS), lambda i, e: (i, 0)),
            pl.BlockSpec((h_sh, Dv), lambda i, e: (0, 0)),
            pl.BlockSpec((Dv, h_sh), lambda i, e: (0, 0)),
            pl.BlockSpec((1, h_ex, Dv), lambda i, e: (e, 0, 0)),
            pl.BlockSpec((1, Dv, h_ex), lambda i, e: (e, 0, 0)),
        ],
        out_specs=pl.BlockSpec((TBLK, Dv), lambda i, e: (i, 0)),
        out_shape=jax.ShapeDtypeStruct((tok, Dv), jnp.float32),
        interpret=interpret,
    )(xf, gates, w1_sh, w2_sh, w1_ex, w2_ex)

    return out.reshape(Bv, Tv, Dv)
